# baseline (speedup 1.0000x reference)
import functools

import jax
import jax.numpy as jnp
from jax import lax
from jax.experimental import pallas as pl
from jax.experimental.pallas import tpu as pltpu
from jax.experimental.pallas import tpu_sc as plsc

NUM_CORES = 2
NUM_SUBCORES = 16
NUM_WORKERS = NUM_CORES * NUM_SUBCORES
LANES = 16
BATCH_N = 16384
FEAT = 64
ROWS_PER_W = BATCH_N // NUM_WORKERS
CHUNK = 128
NCHUNK = ROWS_PER_W // CHUNK


def _body(user_hbm, item_hbm, uemb_hbm, iemb_hbm, out_hbm,
          uidx_v, iidx_v, urows_v, irows_v, out_v, sem):
    wid = lax.axis_index("s") * NUM_CORES + lax.axis_index("c")
    base = wid * ROWS_PER_W

    for j in range(NCHUNK):
        pltpu.sync_copy(user_hbm.at[pl.ds(base + j * CHUNK, CHUNK)],
                        uidx_v.at[j])
        pltpu.sync_copy(item_hbm.at[pl.ds(base + j * CHUNK, CHUNK)],
                        iidx_v.at[j])

    copies = []
    for j in range(NCHUNK):
        copies.append(pltpu.async_copy(
            uemb_hbm.at[uidx_v.at[j]],
            urows_v.at[pl.ds(j * CHUNK, CHUNK)], sem))
        copies.append(pltpu.async_copy(
            iemb_hbm.at[iidx_v.at[j]],
            irows_v.at[pl.ds(j * CHUNK, CHUNK)], sem))
    for c in copies:
        c.wait()

    lane = lax.iota(jnp.int32, LANES)
    last = jnp.full((LANES,), LANES - 1, jnp.int32)

    def blk_body(blk, _):
        acc16 = jnp.zeros((LANES,), jnp.float32)
        for rr in range(LANES):
            r = blk * LANES + rr
            parts = []
            for j in range(FEAT // LANES):
                u = urows_v[r, pl.ds(j * LANES, LANES)]
                i = irows_v[r, pl.ds(j * LANES, LANES)]
                parts.append(u * i)
            s = (parts[0] + parts[1]) + (parts[2] + parts[3])
            tot = jnp.sum(s)
            acc16 = jnp.where(lane == rr, tot, acc16)
        out_v[pl.ds(blk * LANES, LANES)] = acc16
        return ()

    lax.fori_loop(0, ROWS_PER_W // LANES, blk_body, ())

    pltpu.sync_copy(out_v, out_hbm.at[pl.ds(base, ROWS_PER_W)])


@jax.jit
def kernel(user, item, users_emb, items_emb):
    mesh = plsc.VectorSubcoreMesh(core_axis_name="c", subcore_axis_name="s")
    k = pl.kernel(
        _body,
        out_type=jax.ShapeDtypeStruct((BATCH_N,), jnp.float32),
        mesh=mesh,
        scratch_types=[
            pltpu.VMEM((NCHUNK, CHUNK), jnp.int32),
            pltpu.VMEM((NCHUNK, CHUNK), jnp.int32),
            pltpu.VMEM((ROWS_PER_W, FEAT), jnp.float32),
            pltpu.VMEM((ROWS_PER_W, FEAT), jnp.float32),
            pltpu.VMEM((ROWS_PER_W,), jnp.float32),
            pltpu.SemaphoreType.DMA,
        ],
        compiler_params=pltpu.CompilerParams(
            needs_layout_passes=False, use_tc_tiling_on_sc=False),
    )
    return k(user.astype(jnp.int32), item.astype(jnp.int32),
             users_emb, items_emb)
